# baseline (speedup 1.0000x reference)
import jax
import jax.numpy as jnp
import numpy as np
from jax import lax
from jax.experimental import pallas as pl
from jax.experimental.pallas import tpu as pltpu
from jax.experimental.pallas import tpu_sc as plsc

B = 16
N = 100000
H = W = 512
THRESH = 1.0 - 0.03

NP = N

TBL_ROWS = 40
TBL_WORDS = TBL_ROWS * 512
SENTINEL_BASE = H * W

NUM_CORES = 2
NUM_SUBCORES = 16
NUM_WORKERS = NUM_CORES * NUM_SUBCORES
PW = (B * N) // NUM_WORKERS
LANES = 16


def _idx_body(q_ref, out_ref):
    x = q_ref[0]
    y = q_ref[1]
    vx = jnp.round(((x + 1.0) * 512.0 - 1.0) / 2.0)
    vy = jnp.round(((y + 1.0) * 512.0 - 1.0) / 2.0)
    valid = (vx >= 0.0) & (vx <= 511.0) & (vy >= 0.0) & (vy <= 511.0)
    ix = jnp.clip(vx, 0.0, 511.0).astype(jnp.int32)
    iy = jnp.clip(vy, 0.0, 511.0).astype(jnp.int32)
    c = (
        lax.shift_left(lax.shift_right_logical(iy, 4), 13)
        | lax.shift_left(ix, 4)
        | (iy & 15)
    )
    spread = lax.broadcasted_iota(jnp.int32, c.shape, 1) & 4095
    out_ref[...] = jnp.where(valid, c, SENTINEL_BASE + spread)


def _compute_addresses(qp):
    return pl.pallas_call(
        _idx_body,
        grid=(2,),
        in_specs=[
            pl.BlockSpec((2, 8, N), lambda i: (0, i, 0)),
        ],
        out_specs=pl.BlockSpec((8, N), lambda i: (i, 0)),
        out_shape=jax.ShapeDtypeStruct((B, N), jnp.int32),
    )(qp)


def _tbl_body(img_ref, p_ref, tbl_ref):
    t = (img_ref[0] < THRESH).astype(jnp.float32)
    w = jnp.dot(p_ref[...], t, preferred_element_type=jnp.float32)
    wi = w.astype(jnp.int32)
    sent = jnp.full((8, 512), 65535, jnp.int32)
    tbl_ref[0] = jnp.concatenate([wi, sent], axis=0)


def _pack_matrix() -> np.ndarray:
    p = np.zeros((32, 512), np.float32)
    iy = np.arange(512)
    p[iy >> 4, iy] = (1 << (iy & 15)).astype(np.float32)
    return p


def _compute_table(img):
    p = jnp.asarray(_pack_matrix())
    return pl.pallas_call(
        _tbl_body,
        grid=(B,),
        in_specs=[
            pl.BlockSpec((1, H, W), lambda b: (b, 0, 0)),
            pl.BlockSpec((32, 512), lambda b: (0, 0)),
        ],
        out_specs=pl.BlockSpec((1, TBL_ROWS, 512), lambda b: (b, 0, 0)),
        out_shape=jax.ShapeDtypeStruct((B, TBL_ROWS, 512), jnp.int32),
    )(img, p)


UNROLL = 25
STEPS = PW // (LANES * UNROLL)


def _sc_body(tbl_hbm, idx_hbm, out_hbm, tbl_v, idx_v, res_v):
    wid = lax.axis_index("s") * NUM_CORES + lax.axis_index("c")
    img = wid // 2
    base = wid * PW
    pltpu.sync_copy(tbl_hbm.at[img], tbl_v)
    pltpu.sync_copy(idx_hbm.at[pl.ds(base, PW)], idx_v)

    @plsc.parallel_loop(0, PW, LANES, unroll=UNROLL)
    def _loop(s):
        c = idx_v[pl.ds(s, LANES)]
        word = plsc.load_gather(tbl_v, [lax.shift_right_logical(c, 4)])
        bit = lax.bitwise_and(c, 15)
        r = lax.bitwise_and(lax.shift_right_logical(word, bit), 1)
        res_v[pl.ds(s, LANES)] = r

    pltpu.sync_copy(res_v, out_hbm.at[pl.ds(base, PW)])


def _sc_gather(tbl, idx):
    mesh = plsc.VectorSubcoreMesh(core_axis_name="c", subcore_axis_name="s")
    f = pl.kernel(
        _sc_body,
        out_type=jax.ShapeDtypeStruct((B * N,), jnp.int32),
        mesh=mesh,
        scratch_types=[
            pltpu.VMEM((TBL_WORDS,), jnp.int32),
            pltpu.VMEM((PW,), jnp.int32),
            pltpu.VMEM((PW,), jnp.int32),
        ],
        compiler_params=pltpu.CompilerParams(needs_layout_passes=False),
    )
    return f(tbl, idx)


def kernel(query_pts, reference):
    img = reference.reshape(B, H, W)
    qp = jnp.transpose(query_pts, (2, 0, 1))
    idxc = _compute_addresses(qp)
    tbl = _compute_table(img)
    res = _sc_gather(tbl.reshape(B, TBL_WORDS), idxc.reshape(B * N))
    return res.reshape(B, N).astype(jnp.bool_)


# speedup vs baseline: 1.0043x; 1.0043x over previous
import jax
import jax.numpy as jnp
import numpy as np
from jax import lax
from jax.experimental import pallas as pl
from jax.experimental.pallas import tpu as pltpu
from jax.experimental.pallas import tpu_sc as plsc

B = 16
N = 100000
H = W = 512
THRESH = 1.0 - 0.03

NP = N

TBL_ROWS = 40
TBL_WORDS = TBL_ROWS * 512
SENTINEL_BASE = H * W

NUM_CORES = 2
NUM_SUBCORES = 16
NUM_WORKERS = NUM_CORES * NUM_SUBCORES
PW = (B * N) // NUM_WORKERS
LANES = 16


def _idx_body(q_ref, out_ref):
    x = q_ref[0]
    y = q_ref[1]
    vx = jnp.round(((x + 1.0) * 512.0 - 1.0) / 2.0)
    vy = jnp.round(((y + 1.0) * 512.0 - 1.0) / 2.0)
    valid = (vx >= 0.0) & (vx <= 511.0) & (vy >= 0.0) & (vy <= 511.0)
    ix = jnp.clip(vx, 0.0, 511.0).astype(jnp.int32)
    iy = jnp.clip(vy, 0.0, 511.0).astype(jnp.int32)
    c = (
        lax.shift_left(lax.shift_right_logical(iy, 4), 13)
        | lax.shift_left(ix, 4)
        | (iy & 15)
    )
    spread = lax.broadcasted_iota(jnp.int32, c.shape, 1) & 4095
    out_ref[...] = jnp.where(valid, c, SENTINEL_BASE + spread)


def _compute_addresses(qp):
    return pl.pallas_call(
        _idx_body,
        grid=(2,),
        in_specs=[
            pl.BlockSpec((2, 8, N), lambda i: (0, i, 0)),
        ],
        out_specs=pl.BlockSpec((8, N), lambda i: (i, 0)),
        out_shape=jax.ShapeDtypeStruct((B, N), jnp.int32),
    )(qp)


def _tbl_body(img_ref, p_ref, tbl_ref):
    t = (img_ref[0] < THRESH).astype(jnp.float32)
    w = jnp.dot(p_ref[...], t, preferred_element_type=jnp.float32)
    wi = w.astype(jnp.int32)
    sent = jnp.full((8, 512), 65535, jnp.int32)
    tbl_ref[0] = jnp.concatenate([wi, sent], axis=0)


def _pack_matrix() -> np.ndarray:
    p = np.zeros((32, 512), np.float32)
    iy = np.arange(512)
    p[iy >> 4, iy] = (1 << (iy & 15)).astype(np.float32)
    return p


def _compute_table(img):
    p = jnp.asarray(_pack_matrix())
    return pl.pallas_call(
        _tbl_body,
        grid=(B,),
        in_specs=[
            pl.BlockSpec((1, H, W), lambda b: (b, 0, 0)),
            pl.BlockSpec((32, 512), lambda b: (0, 0)),
        ],
        out_specs=pl.BlockSpec((1, TBL_ROWS, 512), lambda b: (b, 0, 0)),
        out_shape=jax.ShapeDtypeStruct((B, TBL_ROWS, 512), jnp.int32),
    )(img, p)


UNROLL = 5
STEPS = PW // (LANES * UNROLL)


def _sc_body(tbl_hbm, idx_hbm, out_hbm, tbl_v, idx_v, res_v):
    wid = lax.axis_index("s") * NUM_CORES + lax.axis_index("c")
    img = wid // 2
    base = wid * PW
    pltpu.sync_copy(tbl_hbm.at[img], tbl_v)
    pltpu.sync_copy(idx_hbm.at[pl.ds(base, PW)], idx_v)

    @plsc.parallel_loop(0, PW, LANES, unroll=UNROLL)
    def _loop(s):
        c = idx_v[pl.ds(s, LANES)]
        word = plsc.load_gather(tbl_v, [lax.shift_right_logical(c, 4)])
        bit = lax.bitwise_and(c, 15)
        r = lax.bitwise_and(lax.shift_right_logical(word, bit), 1)
        res_v[pl.ds(s, LANES)] = r

    pltpu.sync_copy(res_v, out_hbm.at[pl.ds(base, PW)])


def _sc_gather(tbl, idx):
    mesh = plsc.VectorSubcoreMesh(core_axis_name="c", subcore_axis_name="s")
    f = pl.kernel(
        _sc_body,
        out_type=jax.ShapeDtypeStruct((B * N,), jnp.int32),
        mesh=mesh,
        scratch_types=[
            pltpu.VMEM((TBL_WORDS,), jnp.int32),
            pltpu.VMEM((PW,), jnp.int32),
            pltpu.VMEM((PW,), jnp.int32),
        ],
        compiler_params=pltpu.CompilerParams(needs_layout_passes=False),
    )
    return f(tbl, idx)


def kernel(query_pts, reference):
    img = reference.reshape(B, H, W)
    qp = jnp.transpose(query_pts, (2, 0, 1))
    idxc = _compute_addresses(qp)
    tbl = _compute_table(img)
    res = _sc_gather(tbl.reshape(B, TBL_WORDS), idxc.reshape(B * N))
    return res.reshape(B, N).astype(jnp.bool_)
